# baseline (speedup 1.0000x reference)
import jax
import jax.numpy as jnp
import numpy as np
from jax import lax
from jax.experimental import pallas as pl
from jax.experimental.pallas import tpu as pltpu
from jax.experimental.pallas import tpu_sc as plsc

N = 10000
E = 320000
CH = 128
EMB = 64

NPAD = 10240
CHUNK = 128
NBUF = 2
NTILES = 32
CPT = 80
NCHUNKS = NTILES * CPT
EPAD = NCHUNKS * CHUNK
NGRP = CPT // NBUF
ROWS_PT = NPAD // 16


def _mesh():
    return plsc.VectorSubcoreMesh(core_axis_name="c", subcore_axis_name="s")


def _deg_body(dep_hbm, zeros_hbm, out0_hbm, out1_hbm, deg_sh, slab, onesb,
              ssem):
    cid = lax.axis_index("c")
    sid = lax.axis_index("s")
    r0 = sid * ROWS_PT
    pltpu.sync_copy(zeros_hbm.at[pl.ds(r0, ROWS_PT)],
                    deg_sh.at[pl.ds(r0, ROWS_PT)])
    base = (cid * 16 + sid) * CPT
    pltpu.sync_copy(dep_hbm.at[pl.ds(base, CPT)], slab)
    for i in range(CHUNK // 16):
        onesb[pl.ds(i * 16, 16)] = jnp.full((16,), 1.0, jnp.float32)
    plsc.subcore_barrier()

    def fire(c, carry):
        pltpu.async_copy(onesb, deg_sh.at[slab.at[c]], ssem, add=True)
        return carry

    lax.fori_loop(0, CPT, fire, 0)

    def drain(c, carry):
        pltpu.make_async_copy(onesb, deg_sh.at[slab.at[0]], ssem).wait()
        return carry

    lax.fori_loop(0, CPT, drain, 0)
    plsc.subcore_barrier()

    @pl.when(cid == 0)
    def _():
        pltpu.sync_copy(deg_sh.at[pl.ds(r0, ROWS_PT)],
                        out0_hbm.at[pl.ds(r0, ROWS_PT)])

    @pl.when(cid == 1)
    def _():
        pltpu.sync_copy(deg_sh.at[pl.ds(r0, ROWS_PT)],
                        out1_hbm.at[pl.ds(r0, ROWS_PT)])


def _sc_deg(dep, zeros_n):
    return pl.kernel(
        _deg_body,
        out_type=[jax.ShapeDtypeStruct((NPAD,), jnp.float32),
                  jax.ShapeDtypeStruct((NPAD,), jnp.float32)],
        mesh=_mesh(),
        scratch_types=[
            pltpu.VMEM_SHARED((NPAD,), jnp.float32),
            pltpu.VMEM((CPT, CHUNK), jnp.int32),
            pltpu.VMEM((CHUNK,), jnp.float32),
            pltpu.SemaphoreType.DMA,
        ],
    )(dep, zeros_n)


def _agg_body(hs_hbm, sep_hbm, dep_hbm, out0_hbm, out1_hbm, acc_sh, dslab,
              sidx, rows, isems, gsems, ssems):
    cid = lax.axis_index("c")
    sid = lax.axis_index("s")
    r0 = sid * ROWS_PT
    pltpu.sync_copy(hs_hbm.at[pl.ds(r0, ROWS_PT)],
                    acc_sh.at[pl.ds(r0, ROWS_PT)])
    base = (cid * 16 + sid) * CPT
    pltpu.sync_copy(dep_hbm.at[pl.ds(base, CPT)], dslab)
    for b in range(NBUF):
        pltpu.async_copy(sep_hbm.at[base + b], sidx[b], isems[b])
    plsc.subcore_barrier()

    def group(g, carry):
        for b in range(NBUF):
            c = NBUF * g + b
            pltpu.make_async_copy(sep_hbm.at[base], sidx[b], isems[b]).wait()

            @pl.when(g > 0)
            def _():
                pltpu.make_async_copy(rows[b], acc_sh.at[dslab.at[0]],
                                      ssems[b]).wait()
            pltpu.async_copy(hs_hbm.at[sidx[b]], rows[b], gsems[b])
        for b in range(NBUF):
            c = NBUF * g + b
            pltpu.make_async_copy(hs_hbm.at[sidx[b]], rows[b],
                                  gsems[b]).wait()
            nxt = base + lax.min(c + NBUF, CPT - 1)
            pltpu.async_copy(sep_hbm.at[nxt], sidx[b], isems[b])
            pltpu.async_copy(rows[b], acc_sh.at[dslab.at[c]], ssems[b],
                             add=True, priority=1)
        return carry

    lax.fori_loop(0, NGRP, group, 0)
    for b in range(NBUF):
        pltpu.make_async_copy(sep_hbm.at[base], sidx[b], isems[b]).wait()
        pltpu.make_async_copy(rows[b], acc_sh.at[dslab.at[0]],
                              ssems[b]).wait()
    plsc.subcore_barrier()

    @pl.when(cid == 0)
    def _():
        pltpu.sync_copy(acc_sh.at[pl.ds(r0, ROWS_PT)],
                        out0_hbm.at[pl.ds(r0, ROWS_PT)])

    @pl.when(cid == 1)
    def _():
        pltpu.sync_copy(acc_sh.at[pl.ds(r0, ROWS_PT)],
                        out1_hbm.at[pl.ds(r0, ROWS_PT)])


def _sc_agg(hs, sep, dep):
    return pl.kernel(
        _agg_body,
        out_type=[jax.ShapeDtypeStruct((NPAD, CH), jnp.float32),
                  jax.ShapeDtypeStruct((NPAD, CH), jnp.float32)],
        mesh=_mesh(),
        scratch_types=[
            pltpu.VMEM_SHARED((NPAD, CH), jnp.float32),
            pltpu.VMEM((CPT, CHUNK), jnp.int32),
            [pltpu.VMEM((CHUNK,), jnp.int32) for _ in range(NBUF)],
            [pltpu.VMEM((CHUNK, CH), jnp.float32) for _ in range(NBUF)],
            [pltpu.SemaphoreType.DMA for _ in range(NBUF)],
            [pltpu.SemaphoreType.DMA for _ in range(NBUF)],
            [pltpu.SemaphoreType.DMA for _ in range(NBUF)],
        ],
    )(hs, sep, dep)


BR = 1024
GRID = NPAD // BR
BRC = 1024
GRIDC = (N + BRC - 1) // BRC


def _dinv_block(d0_ref, d1_ref):
    deg = d0_ref[...] + d1_ref[...] + 1.0
    return lax.rsqrt(deg)[:, None]


def _ka_body(d0_ref, d1_ref, x_ref, w_ref, hs_ref):
    t = jnp.dot(x_ref[...], w_ref[...], preferred_element_type=jnp.float32)
    hs_ref[...] = t * _dinv_block(d0_ref, d1_ref)


def _tc_first(deg0, deg1, xp, w1):
    return pl.pallas_call(
        _ka_body,
        grid=(GRID,),
        in_specs=[
            pl.BlockSpec((BR,), lambda i: (i,)),
            pl.BlockSpec((BR,), lambda i: (i,)),
            pl.BlockSpec((BR, CH), lambda i: (i, 0)),
            pl.BlockSpec((CH, CH), lambda i: (0, 0)),
        ],
        out_specs=pl.BlockSpec((BR, CH), lambda i: (i, 0)),
        out_shape=jax.ShapeDtypeStruct((NPAD, CH), jnp.float32),
    )(deg0, deg1, xp, w1)


def _kb_body(a0_ref, a1_ref, hs1_ref, d0_ref, d1_ref, b_ref, hs2_ref):
    dinv = _dinv_block(d0_ref, d1_ref)
    s = a0_ref[...] + a1_ref[...] - hs1_ref[...]
    h = jnp.maximum(s * dinv + b_ref[...], 0.0)
    hs2_ref[...] = h * dinv


def _tc_mid(a0, a1, hs1, deg0, deg1, b1):
    return pl.pallas_call(
        _kb_body,
        grid=(GRID,),
        in_specs=[
            pl.BlockSpec((BR, CH), lambda i: (i, 0)),
            pl.BlockSpec((BR, CH), lambda i: (i, 0)),
            pl.BlockSpec((BR, CH), lambda i: (i, 0)),
            pl.BlockSpec((BR,), lambda i: (i,)),
            pl.BlockSpec((BR,), lambda i: (i,)),
            pl.BlockSpec((1, CH), lambda i: (0, 0)),
        ],
        out_specs=pl.BlockSpec((BR, CH), lambda i: (i, 0)),
        out_shape=jax.ShapeDtypeStruct((NPAD, CH), jnp.float32),
    )(a0, a1, hs1, deg0, deg1, b1)


def _kc_body(a0_ref, a1_ref, hs2_ref, d0_ref, d1_ref, wmu_ref, bmu_ref,
             wlv_ref, blv_ref, mu_ref, lv_ref):
    c = (a0_ref[...] + a1_ref[...] - hs2_ref[...]) * _dinv_block(d0_ref,
                                                                 d1_ref)
    mu_ref[...] = (
        jnp.dot(c, wmu_ref[...], preferred_element_type=jnp.float32)
        + bmu_ref[...])
    lv_ref[...] = (
        jnp.dot(c, wlv_ref[...], preferred_element_type=jnp.float32)
        + blv_ref[...])


def _tc_last(a0, a1, hs2, deg0, deg1, wmu, bmu, wlv, blv):
    return pl.pallas_call(
        _kc_body,
        grid=(GRIDC,),
        in_specs=[
            pl.BlockSpec((BRC, CH), lambda i: (i, 0)),
            pl.BlockSpec((BRC, CH), lambda i: (i, 0)),
            pl.BlockSpec((BRC, CH), lambda i: (i, 0)),
            pl.BlockSpec((BRC,), lambda i: (i,)),
            pl.BlockSpec((BRC,), lambda i: (i,)),
            pl.BlockSpec((CH, EMB), lambda i: (0, 0)),
            pl.BlockSpec((1, EMB), lambda i: (0, 0)),
            pl.BlockSpec((CH, EMB), lambda i: (0, 0)),
            pl.BlockSpec((1, EMB), lambda i: (0, 0)),
        ],
        out_specs=[
            pl.BlockSpec((BRC, EMB), lambda i: (i, 0)),
            pl.BlockSpec((BRC, EMB), lambda i: (i, 0)),
        ],
        out_shape=[jax.ShapeDtypeStruct((N, EMB), jnp.float32),
                   jax.ShapeDtypeStruct((N, EMB), jnp.float32)],
    )(a0, a1, hs2, deg0, deg1, wmu, bmu, wlv, blv)


def kernel(x, edges, W1, b1, Wmu, bmu, Wlv, blv):
    src = edges[0]
    dst = edges[1]
    npad_e = EPAD - E
    ar = np.arange(npad_e, dtype=np.int32)
    src_pad = jnp.asarray(ar % 16)
    dst_pad = jnp.asarray(N + (ar % (NPAD - N)), dtype=np.int32)
    sep = jnp.concatenate([src, src_pad]).reshape(NCHUNKS, CHUNK)
    dep = jnp.concatenate([dst, dst_pad]).reshape(NCHUNKS, CHUNK)

    xp = jnp.pad(x, ((0, NPAD - N), (0, 0)))
    zeros_n = jnp.zeros((NPAD,), jnp.float32)
    b1r = b1.reshape(1, CH)
    bmur = bmu.reshape(1, EMB)
    blvr = blv.reshape(1, EMB)

    dep_deg = jnp.pad(dst, (0, EPAD - E), constant_values=N).reshape(
        NCHUNKS, CHUNK)
    deg0, deg1 = _sc_deg(dep_deg, zeros_n)
    hs1 = _tc_first(deg0, deg1, xp, W1)
    a10, a11 = _sc_agg(hs1, sep, dep)
    hs2 = _tc_mid(a10, a11, hs1, deg0, deg1, b1r)
    a20, a21 = _sc_agg(hs2, sep, dep)
    return _tc_last(a20, a21, hs2, deg0, deg1, Wmu, bmur, Wlv, blvr)


# speedup vs baseline: 1.0204x; 1.0204x over previous
import jax
import jax.numpy as jnp
import numpy as np
from jax import lax
from jax.experimental import pallas as pl
from jax.experimental.pallas import tpu as pltpu
from jax.experimental.pallas import tpu_sc as plsc

N = 10000
E = 320000
CH = 128
EMB = 64

NPAD = 10240
CHUNK = 128
NBUF = 2
NTILES = 32
CPT = 80
NCHUNKS = NTILES * CPT
EPAD = NCHUNKS * CHUNK
NGRP = CPT // NBUF
ROWS_PT = NPAD // 16


def _mesh():
    return plsc.VectorSubcoreMesh(core_axis_name="c", subcore_axis_name="s")


def _deg_body(dep_hbm, zeros_hbm, out0_hbm, out1_hbm, deg_sh, slab, onesb,
              ssem):
    cid = lax.axis_index("c")
    sid = lax.axis_index("s")
    r0 = sid * ROWS_PT
    pltpu.sync_copy(zeros_hbm.at[pl.ds(r0, ROWS_PT)],
                    deg_sh.at[pl.ds(r0, ROWS_PT)])
    base = (cid * 16 + sid) * CPT
    pltpu.sync_copy(dep_hbm.at[pl.ds(base, CPT)], slab)
    for i in range(CHUNK // 16):
        onesb[pl.ds(i * 16, 16)] = jnp.full((16,), 1.0, jnp.float32)
    plsc.subcore_barrier()

    def fire(c, carry):
        pltpu.async_copy(onesb, deg_sh.at[slab.at[c]], ssem, add=True)
        return carry

    lax.fori_loop(0, CPT, fire, 0)

    def drain(c, carry):
        pltpu.make_async_copy(onesb, deg_sh.at[slab.at[0]], ssem).wait()
        return carry

    lax.fori_loop(0, CPT, drain, 0)
    plsc.subcore_barrier()

    @pl.when(cid == 0)
    def _():
        pltpu.sync_copy(deg_sh.at[pl.ds(r0, ROWS_PT)],
                        out0_hbm.at[pl.ds(r0, ROWS_PT)])

    @pl.when(cid == 1)
    def _():
        pltpu.sync_copy(deg_sh.at[pl.ds(r0, ROWS_PT)],
                        out1_hbm.at[pl.ds(r0, ROWS_PT)])


def _sc_deg(dep, zeros_n):
    return pl.kernel(
        _deg_body,
        out_type=[jax.ShapeDtypeStruct((NPAD,), jnp.float32),
                  jax.ShapeDtypeStruct((NPAD,), jnp.float32)],
        mesh=_mesh(),
        scratch_types=[
            pltpu.VMEM_SHARED((NPAD,), jnp.float32),
            pltpu.VMEM((CPT, CHUNK), jnp.int32),
            pltpu.VMEM((CHUNK,), jnp.float32),
            pltpu.SemaphoreType.DMA,
        ],
    )(dep, zeros_n)


def _agg_body(hs_hbm, sep_hbm, dep_hbm, out0_hbm, out1_hbm, acc_sh, dslab,
              sidx, rows, isems, gsems, ssems):
    cid = lax.axis_index("c")
    sid = lax.axis_index("s")
    r0 = sid * ROWS_PT
    pltpu.sync_copy(hs_hbm.at[pl.ds(r0, ROWS_PT)],
                    acc_sh.at[pl.ds(r0, ROWS_PT)])
    base = (cid * 16 + sid) * CPT
    pltpu.sync_copy(dep_hbm.at[pl.ds(base, CPT)], dslab)
    for b in range(NBUF):
        pltpu.async_copy(sep_hbm.at[base + b], sidx[b], isems[b])
    plsc.subcore_barrier()

    def group(g, carry):
        for b in range(NBUF):
            c = NBUF * g + b
            pltpu.make_async_copy(sep_hbm.at[base], sidx[b], isems[b]).wait()

            @pl.when(g > 0)
            def _():
                pltpu.make_async_copy(rows[b], acc_sh.at[dslab.at[0]],
                                      ssems[b]).wait()
            pltpu.async_copy(hs_hbm.at[sidx[b]], rows[b], gsems[b])
        for b in range(NBUF):
            c = NBUF * g + b
            pltpu.make_async_copy(hs_hbm.at[sidx[b]], rows[b],
                                  gsems[b]).wait()
            nxt = base + lax.min(c + NBUF, CPT - 1)
            pltpu.async_copy(sep_hbm.at[nxt], sidx[b], isems[b])
            pltpu.async_copy(rows[b], acc_sh.at[dslab.at[c]], ssems[b],
                             add=True)
        return carry

    lax.fori_loop(0, NGRP, group, 0)
    for b in range(NBUF):
        pltpu.make_async_copy(sep_hbm.at[base], sidx[b], isems[b]).wait()
        pltpu.make_async_copy(rows[b], acc_sh.at[dslab.at[0]],
                              ssems[b]).wait()
    plsc.subcore_barrier()

    @pl.when(cid == 0)
    def _():
        pltpu.sync_copy(acc_sh.at[pl.ds(r0, ROWS_PT)],
                        out0_hbm.at[pl.ds(r0, ROWS_PT)])

    @pl.when(cid == 1)
    def _():
        pltpu.sync_copy(acc_sh.at[pl.ds(r0, ROWS_PT)],
                        out1_hbm.at[pl.ds(r0, ROWS_PT)])


def _sc_agg(hs, sep, dep):
    return pl.kernel(
        _agg_body,
        out_type=[jax.ShapeDtypeStruct((NPAD, CH), jnp.float32),
                  jax.ShapeDtypeStruct((NPAD, CH), jnp.float32)],
        mesh=_mesh(),
        scratch_types=[
            pltpu.VMEM_SHARED((NPAD, CH), jnp.float32),
            pltpu.VMEM((CPT, CHUNK), jnp.int32),
            [pltpu.VMEM((CHUNK,), jnp.int32) for _ in range(NBUF)],
            [pltpu.VMEM((CHUNK, CH), jnp.float32) for _ in range(NBUF)],
            [pltpu.SemaphoreType.DMA for _ in range(NBUF)],
            [pltpu.SemaphoreType.DMA for _ in range(NBUF)],
            [pltpu.SemaphoreType.DMA for _ in range(NBUF)],
        ],
    )(hs, sep, dep)


BR = 1024
GRID = NPAD // BR
BRC = 1024
GRIDC = (N + BRC - 1) // BRC


def _dinv_block(d0_ref, d1_ref):
    deg = d0_ref[...] + d1_ref[...] + 1.0
    return lax.rsqrt(deg)[:, None]


def _ka_body(d0_ref, d1_ref, x_ref, w_ref, hs_ref):
    t = jnp.dot(x_ref[...], w_ref[...], preferred_element_type=jnp.float32)
    hs_ref[...] = t * _dinv_block(d0_ref, d1_ref)


def _tc_first(deg0, deg1, xp, w1):
    return pl.pallas_call(
        _ka_body,
        grid=(GRID,),
        in_specs=[
            pl.BlockSpec((BR,), lambda i: (i,)),
            pl.BlockSpec((BR,), lambda i: (i,)),
            pl.BlockSpec((BR, CH), lambda i: (i, 0)),
            pl.BlockSpec((CH, CH), lambda i: (0, 0)),
        ],
        out_specs=pl.BlockSpec((BR, CH), lambda i: (i, 0)),
        out_shape=jax.ShapeDtypeStruct((NPAD, CH), jnp.float32),
    )(deg0, deg1, xp, w1)


def _kb_body(a0_ref, a1_ref, hs1_ref, d0_ref, d1_ref, b_ref, hs2_ref):
    dinv = _dinv_block(d0_ref, d1_ref)
    s = a0_ref[...] + a1_ref[...] - hs1_ref[...]
    h = jnp.maximum(s * dinv + b_ref[...], 0.0)
    hs2_ref[...] = h * dinv


def _tc_mid(a0, a1, hs1, deg0, deg1, b1):
    return pl.pallas_call(
        _kb_body,
        grid=(GRID,),
        in_specs=[
            pl.BlockSpec((BR, CH), lambda i: (i, 0)),
            pl.BlockSpec((BR, CH), lambda i: (i, 0)),
            pl.BlockSpec((BR, CH), lambda i: (i, 0)),
            pl.BlockSpec((BR,), lambda i: (i,)),
            pl.BlockSpec((BR,), lambda i: (i,)),
            pl.BlockSpec((1, CH), lambda i: (0, 0)),
        ],
        out_specs=pl.BlockSpec((BR, CH), lambda i: (i, 0)),
        out_shape=jax.ShapeDtypeStruct((NPAD, CH), jnp.float32),
    )(a0, a1, hs1, deg0, deg1, b1)


def _kc_body(a0_ref, a1_ref, hs2_ref, d0_ref, d1_ref, wmu_ref, bmu_ref,
             wlv_ref, blv_ref, mu_ref, lv_ref):
    c = (a0_ref[...] + a1_ref[...] - hs2_ref[...]) * _dinv_block(d0_ref,
                                                                 d1_ref)
    mu_ref[...] = (
        jnp.dot(c, wmu_ref[...], preferred_element_type=jnp.float32)
        + bmu_ref[...])
    lv_ref[...] = (
        jnp.dot(c, wlv_ref[...], preferred_element_type=jnp.float32)
        + blv_ref[...])


def _tc_last(a0, a1, hs2, deg0, deg1, wmu, bmu, wlv, blv):
    return pl.pallas_call(
        _kc_body,
        grid=(GRIDC,),
        in_specs=[
            pl.BlockSpec((BRC, CH), lambda i: (i, 0)),
            pl.BlockSpec((BRC, CH), lambda i: (i, 0)),
            pl.BlockSpec((BRC, CH), lambda i: (i, 0)),
            pl.BlockSpec((BRC,), lambda i: (i,)),
            pl.BlockSpec((BRC,), lambda i: (i,)),
            pl.BlockSpec((CH, EMB), lambda i: (0, 0)),
            pl.BlockSpec((1, EMB), lambda i: (0, 0)),
            pl.BlockSpec((CH, EMB), lambda i: (0, 0)),
            pl.BlockSpec((1, EMB), lambda i: (0, 0)),
        ],
        out_specs=[
            pl.BlockSpec((BRC, EMB), lambda i: (i, 0)),
            pl.BlockSpec((BRC, EMB), lambda i: (i, 0)),
        ],
        out_shape=[jax.ShapeDtypeStruct((N, EMB), jnp.float32),
                   jax.ShapeDtypeStruct((N, EMB), jnp.float32)],
    )(a0, a1, hs2, deg0, deg1, wmu, bmu, wlv, blv)


def kernel(x, edges, W1, b1, Wmu, bmu, Wlv, blv):
    src = edges[0]
    dst = edges[1]
    npad_e = EPAD - E
    ar = np.arange(npad_e, dtype=np.int32)
    src_pad = jnp.asarray(ar % 16)
    dst_pad = jnp.asarray(N + (ar % (NPAD - N)), dtype=np.int32)
    sep = jnp.concatenate([src, src_pad]).reshape(NCHUNKS, CHUNK)
    dep = jnp.concatenate([dst, dst_pad]).reshape(NCHUNKS, CHUNK)

    xp = jnp.pad(x, ((0, NPAD - N), (0, 0)))
    zeros_n = jnp.zeros((NPAD,), jnp.float32)
    b1r = b1.reshape(1, CH)
    bmur = bmu.reshape(1, EMB)
    blvr = blv.reshape(1, EMB)

    deg0, deg1 = _sc_deg(dep, zeros_n)
    hs1 = _tc_first(deg0, deg1, xp, W1)
    a10, a11 = _sc_agg(hs1, sep, dep)
    hs2 = _tc_mid(a10, a11, hs1, deg0, deg1, b1r)
    a20, a21 = _sc_agg(hs2, sep, dep)
    return _tc_last(a20, a21, hs2, deg0, deg1, Wmu, bmur, Wlv, blvr)
